# Initial kernel scaffold; baseline (speedup 1.0000x reference)
import jax
import jax.numpy as jnp
from jax.experimental import pallas as pl


def kernel(prev, cur, dim, idx):
    raise NotImplementedError("write your pallas kernel here")


# speedup vs baseline: 1.0543x; 1.0543x over previous
import jax
import jax.numpy as jnp
from jax.experimental import pallas as pl
from jax.experimental.pallas import tpu as pltpu


def _body(pos_ref, prev_ref, cur_ref, out_ref):
    pos = pos_ref[0]
    x = prev_ref[...]
    q = cur_ref[...].astype(jnp.float8_e4m3fn).astype(x.dtype)
    row = jax.lax.broadcasted_iota(jnp.int32, x.shape, 1)
    out_ref[...] = jnp.where(row == pos, q, x)


def kernel(prev, cur, dim, idx):
    B, H, S, D = prev.shape
    BH = B * H
    BLK = 8
    prev3 = prev.reshape(BH, S, D)
    cur3 = cur.reshape(BH, 1, D)
    pos = (idx[0] - dim + 1).astype(jnp.int32).reshape((1,))
    grid_spec = pltpu.PrefetchScalarGridSpec(
        num_scalar_prefetch=1,
        grid=(BH // BLK,),
        in_specs=[
            pl.BlockSpec((BLK, S, D), lambda i, pos_ref: (i, 0, 0)),
            pl.BlockSpec((BLK, 1, D), lambda i, pos_ref: (i, 0, 0)),
        ],
        out_specs=pl.BlockSpec((BLK, S, D), lambda i, pos_ref: (i, 0, 0)),
    )
    out = pl.pallas_call(
        _body,
        grid_spec=grid_spec,
        out_shape=jax.ShapeDtypeStruct((BH, S, D), prev.dtype),
    )(pos, prev3, cur3)
    return out.reshape(B, H, S, D)
